# Initial kernel scaffold; baseline (speedup 1.0000x reference)
import jax
import jax.numpy as jnp
from jax.experimental import pallas as pl


def kernel(x, edge_index, theta, Ws2d, Wd2s, Wself, bs2d, bd2s, bself, Wro, bro):
    raise NotImplementedError("write your pallas kernel here")


# speedup vs baseline: 1.0333x; 1.0333x over previous
import functools

import jax
import jax.numpy as jnp
from jax import lax
from jax.experimental import pallas as pl

N = 10000
E = 320000
D = 128


def _dinv(degs):
    def body(d_ref, o_ref):
        dg = d_ref[...]
        o_ref[...] = jnp.where(dg > 0,
                               lax.rsqrt(jnp.maximum(dg, 1e-12)), 0.0)

    return pl.pallas_call(
        body,
        out_shape=jax.ShapeDtypeStruct((4, N), jnp.float32),
    )(degs)


def _dense(af, ab, h, W1, W2, W3, b, Wro, bro, last):
    BN = 400

    def body(af_ref, ab_ref, h_ref, W1_ref, W2_ref, W3_ref, b_ref,
             Wro_ref, bro_ref, o_ref):
        dot = functools.partial(
            jnp.dot, preferred_element_type=jnp.float32,
            precision=lax.Precision.HIGHEST)
        y = (dot(af_ref[...], W1_ref[...])
             + dot(ab_ref[...], W2_ref[...])
             + dot(h_ref[...], W3_ref[...])
             + b_ref[...][None, :])
        if last:
            o_ref[...] = dot(y, Wro_ref[...]) + bro_ref[...][None, :]
        else:
            y = jnp.maximum(y, 0.0)
            nrm = jnp.sqrt(jnp.sum(y * y, axis=1, keepdims=True))
            o_ref[...] = y / jnp.maximum(nrm, 1e-12)

    rows = pl.BlockSpec((BN, D), lambda i: (i, 0))
    full = pl.BlockSpec((D, D), lambda i: (0, 0))
    vec = pl.BlockSpec((D,), lambda i: (0,))
    return pl.pallas_call(
        body,
        grid=(N // BN,),
        in_specs=[rows, rows, rows, full, full, full, vec, full, vec],
        out_specs=rows,
        out_shape=jax.ShapeDtypeStruct((N, D), jnp.float32),
    )(af, ab, h, W1, W2, W3, b, Wro, bro)


def kernel(x, edge_index, theta, Ws2d, Wd2s, Wself, bs2d, bd2s, bself,
           Wro, bro):
    src = edge_index[0]
    dst = edge_index[1]
    t = jax.nn.sigmoid(theta)

    degs = jnp.stack([
        jax.ops.segment_sum(t, src, num_segments=N),
        jax.ops.segment_sum(1.0 - t, src, num_segments=N),
        jax.ops.segment_sum(t, dst, num_segments=N),
        jax.ops.segment_sum(1.0 - t, dst, num_segments=N),
    ])
    dinv = _dinv(degs)
    wf = t * dinv[0][src] * dinv[2][dst]
    wb = (1.0 - t) * dinv[1][src] * dinv[3][dst]

    h = x
    L = Ws2d.shape[0]
    for i in range(L):
        hs = h[src]
        aggf = jax.ops.segment_sum(hs * wf[:, None], dst, num_segments=N)
        aggb = jax.ops.segment_sum(hs * wb[:, None], dst, num_segments=N)
        b = bs2d[i] + bd2s[i] + bself[i]
        h = _dense(aggf, aggb, h, Ws2d[i], Wd2s[i], Wself[i], b,
                   Wro, bro, i == L - 1)
    return h
